# baseline (speedup 1.0000x reference)
import functools

import jax
import jax.numpy as jnp
import numpy as np
from jax import lax
from jax.experimental import pallas as pl
from jax.experimental.pallas import tpu as pltpu
from jax.experimental.pallas import tpu_sc as plsc

N = 10000
E = 320000
F = 128
FH = F // 2
NC = 2
NS = 16
EPT = E // NS
C = 80
NCHT = EPT // C
NGB = 2
NSB = 10
PRE = 2
N_PAD = 10240
ROWS_PT = N_PAD // NS
ZR = 64
LANES = 16

_PERM = np.empty((F,), np.int32)
for _g in range(F // 32):
    _b = 32 * _g
    _PERM[_b + 2 * np.arange(16)] = _b + np.arange(16)
    _PERM[_b + 2 * np.arange(16) + 1] = _b + 16 + np.arange(16)

_mesh = plsc.VectorSubcoreMesh(core_axis_name="c", subcore_axis_name="s",
                               num_cores=NC)


def _buf_types():
    ts = [pltpu.VMEM((NCHT, C), jnp.int32)]
    for _ in range(NGB):
        ts += [pltpu.VMEM((C, FH), jnp.bfloat16),
               pltpu.VMEM((C, F), jnp.float32),
               pltpu.VMEM((C, FH), jnp.float32)]
    for _ in range(NSB):
        ts.append(pltpu.VMEM((C,), jnp.int32))
    ts.append(pltpu.VMEM((ZR, FH), jnp.float32))
    ts.append(pltpu.VMEM_SHARED((N_PAD, FH), jnp.float32))
    for _ in range(NGB):
        ts += [pltpu.SemaphoreType.DMA,
               pltpu.SemaphoreType.DMA,
               pltpu.SemaphoreType.DMA]
    for _ in range(NSB):
        ts.append(pltpu.SemaphoreType.DMA)
    return ts


@functools.partial(
    pl.kernel,
    out_type=jax.ShapeDtypeStruct((NC, N_PAD, FH), jnp.float32),
    mesh=_mesh,
    compiler_params=pltpu.CompilerParams(use_tc_tiling_on_sc=False,
                                         needs_layout_passes=False),
    scratch_types=_buf_types(),
)
def _edge_kernel(f2_hbm, w_hbm, seg_hbm, idx_hbm, out_hbm, idx_v, *scratch):
    o = 0
    rows = [scratch[o + 3 * b] for b in range(NGB)]
    wbuf = [scratch[o + 3 * b + 1] for b in range(NGB)]
    prod = [scratch[o + 3 * b + 2] for b in range(NGB)]
    o += 3 * NGB
    segb = [scratch[o + b] for b in range(NSB)]
    o += NSB
    zbuf = scratch[o]
    accum = scratch[o + 1]
    o += 2
    gsem = [scratch[o + 3 * b] for b in range(NGB)]
    wsem = [scratch[o + 3 * b + 1] for b in range(NGB)]
    scsem = [scratch[o + 3 * b + 2] for b in range(NGB)]
    o += 3 * NGB
    ssem = [scratch[o + b] for b in range(NSB)]

    cid = lax.axis_index("c")
    sid = lax.axis_index("s")

    zero = jnp.zeros((LANES,), jnp.float32)

    def _zero_row(r, _):
        for g in range(FH // LANES):
            zbuf[r, pl.ds(g * LANES, LANES)] = zero
        return 0

    lax.fori_loop(0, ZR, _zero_row, 0)
    base = sid * ROWS_PT
    for k in range(ROWS_PT // ZR):
        pltpu.sync_copy(zbuf, accum.at[pl.ds(base + k * ZR, ZR)])
    plsc.subcore_barrier()

    pltpu.sync_copy(idx_hbm.at[sid], idx_v)

    def _xform(g, _):
        for k in range(C // LANES):
            sl = pl.ds(k * LANES, LANES)
            idx_v[g, sl] = idx_v[g, sl] * 2 + cid
        return 0

    lax.fori_loop(0, NCHT, _xform, 0)

    e_base = sid * EPT

    def _start(g, gb, sb):
        pltpu.async_copy(f2_hbm.at[idx_v.at[g]], rows[gb], gsem[gb])
        pltpu.async_copy(w_hbm.at[pl.ds(e_base + g * C, C)], wbuf[gb],
                         wsem[gb])
        pltpu.async_copy(seg_hbm.at[sid, g], segb[sb], ssem[sb])

    UNROLL = 4
    HMASK = jnp.int32(-65536)

    def _mul_loop(gb, woff):
        rw, wb, pr = rows[gb], wbuf[gb], prod[gb]

        def body(r4, _):
            for u in range(UNROLL):
                r = r4 * UNROLL + u
                for g2 in range(FH // 32):
                    fv = rw[r, pl.ds(32 * g2, 32)]
                    iv = plsc.bitcast(fv, jnp.int32)
                    a = plsc.bitcast(lax.shift_left(iv, 16), jnp.float32)
                    b = plsc.bitcast(iv & HMASK, jnp.float32)
                    sla = pl.ds(32 * g2, LANES)
                    slb = pl.ds(32 * g2 + LANES, LANES)
                    pr[r, sla] = a * wb[r, pl.ds(woff + 32 * g2, LANES)]
                    pr[r, slb] = b * wb[r, pl.ds(woff + 32 * g2 + LANES,
                                                 LANES)]
            return 0
        lax.fori_loop(0, C // UNROLL, body, 0)

    def _mul(gb):
        @pl.when(cid == 0)
        def _():
            _mul_loop(gb, 0)

        @pl.when(cid == 1)
        def _():
            _mul_loop(gb, FH)

    for p in range(PRE):
        _start(p, p % NGB, p % NSB)

    def outer(q, _):
        for j in range(NSB):
            g = q * NSB + j
            gb, sb = j % NGB, j
            pltpu.make_async_copy(f2_hbm.at[idx_v.at[g]], rows[gb],
                                  gsem[gb]).wait()
            pltpu.make_async_copy(w_hbm.at[pl.ds(e_base + g * C, C)],
                                  wbuf[gb], wsem[gb]).wait()
            pltpu.make_async_copy(seg_hbm.at[sid, g], segb[sb],
                                  ssem[sb]).wait()

            @pl.when(g >= NGB)
            def _():
                pltpu.make_async_copy(prod[gb], accum.at[segb[(j - NGB) % NSB]],
                                      scsem[gb]).wait()

            _mul(gb)
            pltpu.async_copy(prod[gb], accum.at[segb[sb]], scsem[gb], add=True)

            @pl.when(g + PRE < NCHT)
            def _():
                _start(g + PRE, (j + PRE) % NGB, (j + PRE) % NSB)
        return 0

    lax.fori_loop(0, NCHT // NSB, outer, 0)

    for t in range(NGB):
        g = NCHT - NGB + t
        pltpu.make_async_copy(prod[g % NGB], accum.at[segb[g % NSB]],
                              scsem[g % NGB]).wait()

    plsc.subcore_barrier()
    pltpu.sync_copy(accum.at[pl.ds(base, ROWS_PT)],
                    out_hbm.at[cid, pl.ds(base, ROWS_PT)])


def _mm_in_body(x_ref, w_ref, o_ref):
    o_ref[...] = jnp.dot(x_ref[...], w_ref[...],
                         preferred_element_type=jnp.float32
                         ).astype(jnp.bfloat16)


def _mm_out_body(p_ref, w_ref, b_ref, o_ref):
    conv = jnp.concatenate([p_ref[0], p_ref[1]], axis=-1)
    o_ref[...] = jnp.dot(conv, w_ref[...],
                         preferred_element_type=jnp.float32) + b_ref[...]


_BLK = 1000


def kernel(x, w_ij, seg_i, idx_j, seg_i_sum, W_in, W_out, b_out):
    f = pl.pallas_call(
        _mm_in_body,
        grid=(N // _BLK,),
        in_specs=[pl.BlockSpec((_BLK, F), lambda i: (i, 0)),
                  pl.BlockSpec((F, F), lambda i: (0, 0))],
        out_specs=pl.BlockSpec((_BLK, F), lambda i: (i, 0)),
        out_shape=jax.ShapeDtypeStruct((N, F), jnp.bfloat16),
    )(x, W_in[:, _PERM])

    f2 = f.reshape(2 * N, FH)
    idx3 = idx_j.astype(jnp.int32).reshape(NS, NCHT, C)
    seg3 = seg_i.astype(jnp.int32).reshape(NS, NCHT, C)

    partials = _edge_kernel(f2, w_ij, seg3, idx3)

    bias = (b_out
            + (jnp.asarray(seg_i_sum, jnp.float32) - jnp.float32(N))
            ).reshape(1, F)

    c = pl.pallas_call(
        _mm_out_body,
        grid=(N // _BLK,),
        in_specs=[pl.BlockSpec((NC, _BLK, FH), lambda i: (0, i, 0)),
                  pl.BlockSpec((F, F), lambda i: (0, 0)),
                  pl.BlockSpec((1, F), lambda i: (0, 0))],
        out_specs=pl.BlockSpec((_BLK, F), lambda i: (i, 0)),
        out_shape=jax.ShapeDtypeStruct((N, F), jnp.float32),
    )(partials, W_out, bias)
    return c


# speedup vs baseline: 1.5524x; 1.5524x over previous
import functools

import jax
import jax.numpy as jnp
import numpy as np
from jax import lax
from jax.experimental import pallas as pl
from jax.experimental.pallas import tpu as pltpu
from jax.experimental.pallas import tpu_sc as plsc

N = 10000
E = 320000
F = 128
FH = F // 2
NC = 2
NS = 16
EPT = E // NS
C = 80
NCHT = EPT // C
NGB = 5
NPB = 2
NSB = 10
PRE = 4
N_PAD = 10240
ROWS_PT = N_PAD // NS
ZR = 64
LANES = 16

_PERM = np.empty((F,), np.int32)
for _g in range(F // 32):
    _b = 32 * _g
    _PERM[_b + 2 * np.arange(16)] = _b + np.arange(16)
    _PERM[_b + 2 * np.arange(16) + 1] = _b + 16 + np.arange(16)

_mesh = plsc.VectorSubcoreMesh(core_axis_name="c", subcore_axis_name="s",
                               num_cores=NC)


def _buf_types():
    ts = [pltpu.VMEM((NCHT, C), jnp.int32)]
    for _ in range(NGB):
        ts += [pltpu.VMEM((C, FH), jnp.bfloat16),
               pltpu.VMEM((C, FH), jnp.float32),
               pltpu.VMEM((C,), jnp.int32)]
    for _ in range(NSB):
        ts.append(pltpu.VMEM((C,), jnp.int32))
    for _ in range(NPB):
        ts.append(pltpu.VMEM((C, FH), jnp.float32))
    ts.append(pltpu.VMEM((ZR, FH), jnp.float32))
    ts.append(pltpu.VMEM_SHARED((N_PAD, FH), jnp.float32))
    for _ in range(NGB):
        ts += [pltpu.SemaphoreType.DMA,
               pltpu.SemaphoreType.DMA]
    for _ in range(NSB):
        ts.append(pltpu.SemaphoreType.DMA)
    for _ in range(NPB):
        ts.append(pltpu.SemaphoreType.DMA)
    return ts


@functools.partial(
    pl.kernel,
    out_type=jax.ShapeDtypeStruct((NC, N_PAD, FH), jnp.float32),
    mesh=_mesh,
    compiler_params=pltpu.CompilerParams(use_tc_tiling_on_sc=False,
                                         needs_layout_passes=False),
    scratch_types=_buf_types(),
)
def _edge_kernel(f2_hbm, w2_hbm, seg_hbm, idx_hbm, out_hbm, idx_v, *scratch):
    o = 0
    rows = [scratch[o + 3 * b] for b in range(NGB)]
    wbuf = [scratch[o + 3 * b + 1] for b in range(NGB)]
    widx = [scratch[o + 3 * b + 2] for b in range(NGB)]
    o += 3 * NGB
    segb = [scratch[o + b] for b in range(NSB)]
    o += NSB
    prod = [scratch[o + b] for b in range(NPB)]
    o += NPB
    zbuf = scratch[o]
    accum = scratch[o + 1]
    o += 2
    gsem = [scratch[o + 2 * b] for b in range(NGB)]
    wsem = [scratch[o + 2 * b + 1] for b in range(NGB)]
    o += 2 * NGB
    ssem = [scratch[o + b] for b in range(NSB)]
    o += NSB
    scsem = [scratch[o + b] for b in range(NPB)]

    cid = lax.axis_index("c")
    sid = lax.axis_index("s")

    zero = jnp.zeros((LANES,), jnp.float32)

    def _zero_row(r, _):
        for g in range(FH // LANES):
            zbuf[r, pl.ds(g * LANES, LANES)] = zero
        return 0

    lax.fori_loop(0, ZR, _zero_row, 0)
    base = sid * ROWS_PT
    for k in range(ROWS_PT // ZR):
        pltpu.sync_copy(zbuf, accum.at[pl.ds(base + k * ZR, ZR)])
    plsc.subcore_barrier()

    pltpu.sync_copy(idx_hbm.at[sid], idx_v)

    two_iota = lax.iota(jnp.int32, LANES) * 2

    def _xform(g, _):
        for k in range(C // LANES):
            sl = pl.ds(k * LANES, LANES)
            idx_v[g, sl] = idx_v[g, sl] * 2 + cid
        return 0

    lax.fori_loop(0, NCHT, _xform, 0)

    e_base2 = (sid * EPT) * 2 + cid

    def _start(g, gb, sb):
        e0 = e_base2 + g * (2 * C)
        for k in range(C // LANES):
            widx[gb][pl.ds(k * LANES, LANES)] = two_iota + (e0 + 2 * k * LANES)
        pltpu.async_copy(f2_hbm.at[idx_v.at[g]], rows[gb], gsem[gb])
        pltpu.async_copy(w2_hbm.at[widx[gb]], wbuf[gb], wsem[gb])
        pltpu.async_copy(seg_hbm.at[sid, g], segb[sb], ssem[sb])

    UNROLL = 4
    HMASK = jnp.int32(-65536)

    def _mul(gb, pb):
        rw, wb, pr = rows[gb], wbuf[gb], prod[pb]

        def body(r4, _):
            for u in range(UNROLL):
                r = r4 * UNROLL + u
                for g2 in range(FH // 32):
                    fv = rw[r, pl.ds(32 * g2, 32)]
                    iv = plsc.bitcast(fv, jnp.int32)
                    a = plsc.bitcast(lax.shift_left(iv, 16), jnp.float32)
                    b = plsc.bitcast(iv & HMASK, jnp.float32)
                    sla = pl.ds(32 * g2, LANES)
                    slb = pl.ds(32 * g2 + LANES, LANES)
                    pr[r, sla] = a * wb[r, sla]
                    pr[r, slb] = b * wb[r, slb]
            return 0
        lax.fori_loop(0, C // UNROLL, body, 0)

    for p in range(PRE):
        _start(p, p % NGB, p % NSB)

    def outer(q, _):
        for j in range(NSB):
            g = q * NSB + j
            gb, pb, sb = j % NGB, j % NPB, j
            pltpu.make_async_copy(f2_hbm.at[idx_v.at[g]], rows[gb],
                                  gsem[gb]).wait()
            pltpu.make_async_copy(w2_hbm.at[widx[gb]], wbuf[gb],
                                  wsem[gb]).wait()
            pltpu.make_async_copy(seg_hbm.at[sid, g], segb[sb],
                                  ssem[sb]).wait()

            @pl.when(g >= NPB)
            def _():
                pltpu.make_async_copy(prod[pb], accum.at[segb[(j - NPB) % NSB]],
                                      scsem[pb]).wait()

            _mul(gb, pb)
            pltpu.async_copy(prod[pb], accum.at[segb[sb]], scsem[pb], add=True)

            @pl.when(g + PRE < NCHT)
            def _():
                _start(g + PRE, (j + PRE) % NGB, (j + PRE) % NSB)
        return 0

    lax.fori_loop(0, NCHT // NSB, outer, 0)

    for t in range(NPB):
        g = NCHT - NPB + t
        pltpu.make_async_copy(prod[g % NPB], accum.at[segb[g % NSB]],
                              scsem[g % NPB]).wait()

    plsc.subcore_barrier()
    pltpu.sync_copy(accum.at[pl.ds(base, ROWS_PT)],
                    out_hbm.at[cid, pl.ds(base, ROWS_PT)])


def _mm_in_body(x_ref, w_ref, o_ref):
    o_ref[...] = jnp.dot(x_ref[...], w_ref[...],
                         preferred_element_type=jnp.float32
                         ).astype(jnp.bfloat16)


def _mm_out_body(p_ref, w_ref, b_ref, o_ref):
    conv = jnp.concatenate([p_ref[0], p_ref[1]], axis=-1)
    o_ref[...] = jnp.dot(conv, w_ref[...],
                         preferred_element_type=jnp.float32) + b_ref[...]


_BLK = 1000


def kernel(x, w_ij, seg_i, idx_j, seg_i_sum, W_in, W_out, b_out):
    f = pl.pallas_call(
        _mm_in_body,
        grid=(N // _BLK,),
        in_specs=[pl.BlockSpec((_BLK, F), lambda i: (i, 0)),
                  pl.BlockSpec((F, F), lambda i: (0, 0))],
        out_specs=pl.BlockSpec((_BLK, F), lambda i: (i, 0)),
        out_shape=jax.ShapeDtypeStruct((N, F), jnp.bfloat16),
    )(x, W_in[:, _PERM])

    f2 = f.reshape(2 * N, FH)
    w2 = w_ij.reshape(2 * E, FH)
    idx3 = idx_j.astype(jnp.int32).reshape(NS, NCHT, C)
    seg3 = seg_i.astype(jnp.int32).reshape(NS, NCHT, C)

    partials = _edge_kernel(f2, w2, seg3, idx3)

    bias = (b_out
            + (jnp.asarray(seg_i_sum, jnp.float32) - jnp.float32(N))
            ).reshape(1, F)

    c = pl.pallas_call(
        _mm_out_body,
        grid=(N // _BLK,),
        in_specs=[pl.BlockSpec((NC, _BLK, FH), lambda i: (0, i, 0)),
                  pl.BlockSpec((F, F), lambda i: (0, 0)),
                  pl.BlockSpec((1, F), lambda i: (0, 0))],
        out_specs=pl.BlockSpec((_BLK, F), lambda i: (i, 0)),
        out_shape=jax.ShapeDtypeStruct((N, F), jnp.float32),
    )(partials, W_out, bias)
    return c


# speedup vs baseline: 2.1718x; 1.3990x over previous
import functools

import jax
import jax.numpy as jnp
from jax import lax
from jax.experimental import pallas as pl
from jax.experimental.pallas import tpu as pltpu
from jax.experimental.pallas import tpu_sc as plsc

N = 10000
E = 320000
F = 128
FH = F // 2
NC = 2
NS = 16
EPT = E // NS
C = 80
NCHT = EPT // C
NBUF = 5
PRE = NBUF - 2
N_PAD = 10240
ROWS_PT = N_PAD // NS
ZR = 64
LANES = 16

_mesh = plsc.VectorSubcoreMesh(core_axis_name="c", subcore_axis_name="s",
                               num_cores=NC)


def _buf_types():
    ts = [pltpu.VMEM((NCHT, C), jnp.int32)]
    for _ in range(NBUF):
        ts += [pltpu.VMEM((C, FH), jnp.float32),
               pltpu.VMEM((C, FH), jnp.float32),
               pltpu.VMEM((C,), jnp.int32),
               pltpu.VMEM((C,), jnp.int32)]
    ts.append(pltpu.VMEM((ZR, FH), jnp.float32))
    ts.append(pltpu.VMEM_SHARED((N_PAD, FH), jnp.float32))
    for _ in range(NBUF):
        ts += [pltpu.SemaphoreType.DMA,
               pltpu.SemaphoreType.DMA,
               pltpu.SemaphoreType.DMA,
               pltpu.SemaphoreType.DMA]
    return ts


@functools.partial(
    pl.kernel,
    out_type=jax.ShapeDtypeStruct((NC, N_PAD, FH), jnp.float32),
    mesh=_mesh,
    compiler_params=pltpu.CompilerParams(use_tc_tiling_on_sc=False),
    scratch_types=_buf_types(),
)
def _edge_kernel(f2_hbm, w2_hbm, seg_hbm, idx_hbm, out_hbm, idx_v, *scratch):
    bufs = tuple(scratch[b * 4:b * 4 + 4] for b in range(NBUF))
    zbuf = scratch[NBUF * 4]
    accum = scratch[NBUF * 4 + 1]
    sems = tuple(scratch[NBUF * 4 + 2 + b * 4:NBUF * 4 + 6 + b * 4]
                 for b in range(NBUF))

    cid = lax.axis_index("c")
    sid = lax.axis_index("s")

    zero = jnp.zeros((LANES,), jnp.float32)

    def _zero_row(r, _):
        for g in range(FH // LANES):
            zbuf[r, pl.ds(g * LANES, LANES)] = zero
        return 0

    lax.fori_loop(0, ZR, _zero_row, 0)
    base = sid * ROWS_PT
    for k in range(ROWS_PT // ZR):
        pltpu.sync_copy(zbuf, accum.at[pl.ds(base + k * ZR, ZR)])
    plsc.subcore_barrier()

    pltpu.sync_copy(idx_hbm.at[sid], idx_v)

    two_iota = lax.iota(jnp.int32, LANES) * 2

    def _xform(g, _):
        for k in range(C // LANES):
            sl = pl.ds(k * LANES, LANES)
            idx_v[g, sl] = idx_v[g, sl] * 2 + cid
        return 0

    lax.fori_loop(0, NCHT, _xform, 0)

    e_base2 = (sid * EPT) * 2 + cid

    def _start(g, b):
        rows, w, wi, segb = bufs[b]
        gsem, wsem, ssem, _ = sems[b]
        e0 = e_base2 + g * (2 * C)
        for k in range(C // LANES):
            wi[pl.ds(k * LANES, LANES)] = two_iota + (e0 + 2 * k * LANES)
        pltpu.async_copy(f2_hbm.at[idx_v.at[g]], rows, gsem)
        pltpu.async_copy(w2_hbm.at[wi], w, wsem)
        pltpu.async_copy(seg_hbm.at[sid, g], segb, ssem)

    UNROLL = 4

    def _mul(rows, w):
        def body(r4, _):
            for u in range(UNROLL):
                for g in range(FH // LANES):
                    sl = pl.ds(g * LANES, LANES)
                    r = r4 * UNROLL + u
                    rows[r, sl] = rows[r, sl] * w[r, sl]
            return 0
        lax.fori_loop(0, C // UNROLL, body, 0)

    for p in range(PRE):
        _start(p, p)

    def outer(q, _):
        for b in range(NBUF):
            g = q * NBUF + b
            rows, w, wi, segb = bufs[b]
            gsem, wsem, ssem, scsem = sems[b]
            pltpu.make_async_copy(f2_hbm.at[idx_v.at[g]], rows, gsem).wait()
            pltpu.make_async_copy(w2_hbm.at[wi], w, wsem).wait()
            pltpu.make_async_copy(seg_hbm.at[sid, g], segb, ssem).wait()
            _mul(rows, w)
            pltpu.async_copy(rows, accum.at[segb], scsem, add=True)

            nb = (b + PRE) % NBUF
            nrows, _, _, nsegb = bufs[nb]
            nscsem = sems[nb][3]

            @pl.when(g + PRE < NCHT)
            def _():
                @pl.when(g > 1)
                def _():
                    pltpu.make_async_copy(
                        nrows, accum.at[nsegb], nscsem).wait()
                _start(g + PRE, nb)
        return 0

    lax.fori_loop(0, NCHT // NBUF, outer, 0)

    for b in range(NBUF):
        rows, _, _, segb = bufs[b]
        scsem = sems[b][3]
        pltpu.make_async_copy(rows, accum.at[segb], scsem).wait()

    plsc.subcore_barrier()
    pltpu.sync_copy(accum.at[pl.ds(base, ROWS_PT)],
                    out_hbm.at[cid, pl.ds(base, ROWS_PT)])


def _mm_in_body(x_ref, w_ref, o_ref):
    o_ref[...] = jnp.dot(x_ref[...], w_ref[...],
                         preferred_element_type=jnp.float32)


def _mm_out_body(p_ref, w_ref, b_ref, o_ref):
    conv = jnp.concatenate([p_ref[0], p_ref[1]], axis=-1)
    o_ref[...] = jnp.dot(conv, w_ref[...],
                         preferred_element_type=jnp.float32) + b_ref[...]


_BLK = 1000


def kernel(x, w_ij, seg_i, idx_j, seg_i_sum, W_in, W_out, b_out):
    f = pl.pallas_call(
        _mm_in_body,
        grid=(N // _BLK,),
        in_specs=[pl.BlockSpec((_BLK, F), lambda i: (i, 0)),
                  pl.BlockSpec((F, F), lambda i: (0, 0))],
        out_specs=pl.BlockSpec((_BLK, F), lambda i: (i, 0)),
        out_shape=jax.ShapeDtypeStruct((N, F), jnp.float32),
    )(x, W_in)

    f2 = f.reshape(2 * N, FH)
    w2 = w_ij.reshape(2 * E, FH)
    idx3 = idx_j.astype(jnp.int32).reshape(NS, NCHT, C)
    seg3 = seg_i.astype(jnp.int32).reshape(NS, NCHT, C)

    partials = _edge_kernel(f2, w2, seg3, idx3)

    bias = (b_out
            + (jnp.asarray(seg_i_sum, jnp.float32) - jnp.float32(N))
            ).reshape(1, F)

    c = pl.pallas_call(
        _mm_out_body,
        grid=(N // _BLK,),
        in_specs=[pl.BlockSpec((NC, _BLK, FH), lambda i: (0, i, 0)),
                  pl.BlockSpec((F, F), lambda i: (0, 0)),
                  pl.BlockSpec((1, F), lambda i: (0, 0))],
        out_specs=pl.BlockSpec((_BLK, F), lambda i: (i, 0)),
        out_shape=jax.ShapeDtypeStruct((N, F), jnp.float32),
    )(partials, W_out, bias)
    return c


# speedup vs baseline: 2.1999x; 1.0130x over previous
import functools

import jax
import jax.numpy as jnp
from jax import lax
from jax.experimental import pallas as pl
from jax.experimental.pallas import tpu as pltpu
from jax.experimental.pallas import tpu_sc as plsc

N = 10000
E = 320000
F = 128
FH = F // 2
NC = 2
NS = 16
EPT = E // NS
C = 80
NCHT = EPT // C
NBUF = 5
PRE = NBUF - 1
N_PAD = 10240
ROWS_PT = N_PAD // NS
ZR = 64
LANES = 16

_mesh = plsc.VectorSubcoreMesh(core_axis_name="c", subcore_axis_name="s",
                               num_cores=NC)


def _buf_types():
    ts = [pltpu.VMEM((NCHT, C), jnp.int32)]
    for _ in range(NBUF):
        ts += [pltpu.VMEM((C, FH), jnp.float32),
               pltpu.VMEM((C, FH), jnp.float32),
               pltpu.VMEM((C,), jnp.int32),
               pltpu.VMEM((C,), jnp.int32)]
    ts.append(pltpu.VMEM((ZR, FH), jnp.float32))
    ts.append(pltpu.VMEM_SHARED((N_PAD, FH), jnp.float32))
    for _ in range(NBUF):
        ts += [pltpu.SemaphoreType.DMA,
               pltpu.SemaphoreType.DMA,
               pltpu.SemaphoreType.DMA,
               pltpu.SemaphoreType.DMA]
    return ts


@functools.partial(
    pl.kernel,
    out_type=jax.ShapeDtypeStruct((NC, N_PAD, FH), jnp.float32),
    mesh=_mesh,
    compiler_params=pltpu.CompilerParams(use_tc_tiling_on_sc=False),
    scratch_types=_buf_types(),
)
def _edge_kernel(f2_hbm, w2_hbm, seg_hbm, idx_hbm, out_hbm, idx_v, *scratch):
    bufs = tuple(scratch[b * 4:b * 4 + 4] for b in range(NBUF))
    zbuf = scratch[NBUF * 4]
    accum = scratch[NBUF * 4 + 1]
    sems = tuple(scratch[NBUF * 4 + 2 + b * 4:NBUF * 4 + 6 + b * 4]
                 for b in range(NBUF))

    cid = lax.axis_index("c")
    sid = lax.axis_index("s")

    zero = jnp.zeros((LANES,), jnp.float32)

    def _zero_row(r, _):
        for g in range(FH // LANES):
            zbuf[r, pl.ds(g * LANES, LANES)] = zero
        return 0

    lax.fori_loop(0, ZR, _zero_row, 0)
    base = sid * ROWS_PT
    for k in range(ROWS_PT // ZR):
        pltpu.sync_copy(zbuf, accum.at[pl.ds(base + k * ZR, ZR)])
    plsc.subcore_barrier()

    pltpu.sync_copy(idx_hbm.at[sid], idx_v)

    two_iota = lax.iota(jnp.int32, LANES) * 2

    def _xform(g, _):
        for k in range(C // LANES):
            sl = pl.ds(k * LANES, LANES)
            idx_v[g, sl] = idx_v[g, sl] * 2 + cid
        return 0

    lax.fori_loop(0, NCHT, _xform, 0)

    e_base2 = (sid * EPT) * 2 + cid

    def _start(g, b):
        rows, w, wi, segb = bufs[b]
        gsem, wsem, ssem, _ = sems[b]
        e0 = e_base2 + g * (2 * C)
        for k in range(C // LANES):
            wi[pl.ds(k * LANES, LANES)] = two_iota + (e0 + 2 * k * LANES)
        pltpu.async_copy(f2_hbm.at[idx_v.at[g]], rows, gsem)
        pltpu.async_copy(w2_hbm.at[wi], w, wsem)
        pltpu.async_copy(seg_hbm.at[sid, g], segb, ssem)

    UNROLL = 4

    def _mul(rows, w):
        def body(r4, _):
            for u in range(UNROLL):
                for g in range(FH // LANES):
                    sl = pl.ds(g * LANES, LANES)
                    r = r4 * UNROLL + u
                    rows[r, sl] = rows[r, sl] * w[r, sl]
            return 0
        lax.fori_loop(0, C // UNROLL, body, 0)

    for p in range(PRE):
        _start(p, p)

    def outer(q, _):
        for b in range(NBUF):
            g = q * NBUF + b
            rows, w, wi, segb = bufs[b]
            gsem, wsem, ssem, scsem = sems[b]
            pltpu.make_async_copy(f2_hbm.at[idx_v.at[g]], rows, gsem).wait()
            pltpu.make_async_copy(w2_hbm.at[wi], w, wsem).wait()
            pltpu.make_async_copy(seg_hbm.at[sid, g], segb, ssem).wait()
            _mul(rows, w)
            pltpu.async_copy(rows, accum.at[segb], scsem, add=True)

            nb = (b + PRE) % NBUF
            nrows, _, _, nsegb = bufs[nb]
            nscsem = sems[nb][3]

            @pl.when(g + PRE < NCHT)
            def _():
                @pl.when(g > 0)
                def _():
                    pltpu.make_async_copy(
                        nrows, accum.at[nsegb], nscsem).wait()
                _start(g + PRE, nb)
        return 0

    lax.fori_loop(0, NCHT // NBUF, outer, 0)

    for b in range(NBUF):
        rows, _, _, segb = bufs[b]
        scsem = sems[b][3]
        pltpu.make_async_copy(rows, accum.at[segb], scsem).wait()

    plsc.subcore_barrier()
    pltpu.sync_copy(accum.at[pl.ds(base, ROWS_PT)],
                    out_hbm.at[cid, pl.ds(base, ROWS_PT)])


def _mm_in_body(x_ref, w_ref, o_ref):
    o_ref[...] = jnp.dot(x_ref[...], w_ref[...],
                         preferred_element_type=jnp.float32)


def _mm_out_body(p_ref, w_ref, b_ref, o_ref):
    conv = jnp.concatenate([p_ref[0], p_ref[1]], axis=-1)
    o_ref[...] = jnp.dot(conv, w_ref[...],
                         preferred_element_type=jnp.float32) + b_ref[...]


_BLK = 1000


def kernel(x, w_ij, seg_i, idx_j, seg_i_sum, W_in, W_out, b_out):
    f = pl.pallas_call(
        _mm_in_body,
        grid=(N // _BLK,),
        in_specs=[pl.BlockSpec((_BLK, F), lambda i: (i, 0)),
                  pl.BlockSpec((F, F), lambda i: (0, 0))],
        out_specs=pl.BlockSpec((_BLK, F), lambda i: (i, 0)),
        out_shape=jax.ShapeDtypeStruct((N, F), jnp.float32),
    )(x, W_in)

    f2 = f.reshape(2 * N, FH)
    w2 = w_ij.reshape(2 * E, FH)
    idx3 = idx_j.astype(jnp.int32).reshape(NS, NCHT, C)
    seg3 = seg_i.astype(jnp.int32).reshape(NS, NCHT, C)

    partials = _edge_kernel(f2, w2, seg3, idx3)

    bias = (b_out
            + (jnp.asarray(seg_i_sum, jnp.float32) - jnp.float32(N))
            ).reshape(1, F)

    c = pl.pallas_call(
        _mm_out_body,
        grid=(N // _BLK,),
        in_specs=[pl.BlockSpec((NC, _BLK, FH), lambda i: (0, i, 0)),
                  pl.BlockSpec((F, F), lambda i: (0, 0)),
                  pl.BlockSpec((1, F), lambda i: (0, 0))],
        out_specs=pl.BlockSpec((_BLK, F), lambda i: (i, 0)),
        out_shape=jax.ShapeDtypeStruct((N, F), jnp.float32),
    )(partials, W_out, bias)
    return c
